# baseline (speedup 1.0000x reference)
import functools

import jax
import jax.numpy as jnp
import numpy as np
from jax import lax
from jax.experimental import pallas as pl
from jax.experimental.pallas import tpu as pltpu
from jax.experimental.pallas import tpu_sc as plsc

VOCAB = 100000
EMB_DIM = 128
GROUP_SIZE = 32
GROUPS = EMB_DIM // GROUP_SIZE
WORDS = EMB_DIM // 4

NUM_CORES = 2
NUM_SUBCORES = 16
NUM_WORKERS = NUM_CORES * NUM_SUBCORES

CHUNK = 512


def _column_perm() -> np.ndarray:
    perm = np.empty(EMB_DIM, dtype=np.int32)
    for p in range(EMB_DIM):
        w, b = p >> 2, p & 3
        c, l = w >> 4, w & 15
        m, o = b >> 1, b & 1
        perm[p] = 64 * c + 32 * m + 2 * l + o
    return perm


_PERM = _column_perm()


def _f16_bits(p):
    b = plsc.bitcast(p, jnp.int32)
    m = b & jnp.int32(0x7FFFFFFF)
    lsb = lax.shift_right_logical(m, jnp.int32(13)) & jnp.int32(1)
    r = (m + lsb) + jnp.int32(0xFFF - 0x38000000)
    h = lax.shift_right_logical(r, jnp.int32(13))
    h = jnp.where(m < jnp.int32(0x38800000), jnp.int32(0), h)
    s = lax.shift_right_logical(b, jnp.int32(16)) & jnp.int32(0x8000)
    return s | h


def _extract_byte(v, b):
    if b == 3:
        return lax.shift_right_arithmetic(v, jnp.int32(24))
    return lax.shift_right_arithmetic(
        lax.shift_left(v, jnp.int32(24 - 8 * b)), jnp.int32(24)
    )


def _make_kernel(n_total: int):
    per_worker = n_total // NUM_WORKERS
    n_chunks = per_worker // CHUNK
    mesh = plsc.VectorSubcoreMesh(core_axis_name="c", subcore_axis_name="s")

    @functools.partial(
        pl.kernel,
        out_type=jax.ShapeDtypeStruct((n_total, WORDS * 2), jnp.int32),
        mesh=mesh,
        scratch_types=[
            pltpu.VMEM((CHUNK,), jnp.int32),
            pltpu.VMEM((CHUNK * GROUPS,), jnp.int32),
            pltpu.VMEM((CHUNK, WORDS), jnp.int32),
            pltpu.VMEM((CHUNK * GROUPS,), jnp.float32),
            pltpu.VMEM((CHUNK, WORDS * 2), jnp.int32),
            pltpu.SemaphoreType.DMA,
            pltpu.SemaphoreType.DMA,
        ],
        compiler_params=pltpu.CompilerParams(
            needs_layout_passes=False, use_tc_tiling_on_sc=False
        ),
    )
    def k(
        idx_hbm, idx4_hbm, w_hbm, s_hbm, out_hbm,
        idx_v, idx4_v, w_v, s_v, o_v, sem_w, sem_s,
    ):
        wid = lax.axis_index("s") * NUM_CORES + lax.axis_index("c")
        base = wid * per_worker

        def chunk_body(ci, _):
            cb = base + ci * CHUNK
            pltpu.sync_copy(idx_hbm.at[pl.ds(cb, CHUNK)], idx_v)
            pltpu.sync_copy(
                idx4_hbm.at[pl.ds(cb * GROUPS, CHUNK * GROUPS)], idx4_v
            )
            cw = pltpu.async_copy(w_hbm.at[idx_v], w_v, sem_w)
            cs = pltpu.async_copy(s_hbm.at[idx4_v], s_v, sem_s)
            cw.wait()
            cs.wait()

            def row_body(i, _):
                srow = jnp.full((16,), i * GROUPS, jnp.int32)
                w0 = w_v[i, pl.ds(0, 16)]
                w1 = w_v[i, pl.ds(16, 16)]
                for g in range(GROUPS):
                    vreg = w0 if g < 2 else w1
                    m = g & 1
                    ev = _extract_byte(vreg, 2 * m).astype(jnp.float32)
                    od = _extract_byte(vreg, 2 * m + 1).astype(jnp.float32)
                    sc = plsc.load_gather(s_v, [srow + jnp.int32(g)])
                    ei = _f16_bits(ev * sc)
                    oi = _f16_bits(od * sc)
                    words = ei | lax.shift_left(oi, jnp.int32(16))
                    o_v[i, pl.ds(16 * g, 16)] = words
                return 0

            lax.fori_loop(0, CHUNK, row_body, 0)
            pltpu.sync_copy(o_v, out_hbm.at[pl.ds(cb, CHUNK)])
            return 0

        lax.fori_loop(0, n_chunks, chunk_body, 0)

    return k


@jax.jit
def kernel(indices, weight, scales):
    b, l = indices.shape
    n = b * l
    idx_flat = indices.reshape(n).astype(jnp.int32)
    w_perm = weight.reshape(VOCAB, 2, 2, 16, 2).transpose(0, 1, 3, 2, 4)
    w_i32 = lax.bitcast_convert_type(
        w_perm.reshape(VOCAB, WORDS, 4), jnp.int32
    )
    s_f32 = scales.astype(jnp.float32).reshape(VOCAB * GROUPS)
    idx4 = (idx_flat[:, None] * GROUPS + jnp.arange(GROUPS, dtype=jnp.int32)
            ).reshape(n * GROUPS)
    out = _make_kernel(n)(idx_flat, idx4, w_i32, s_f32)
    out_f16 = lax.bitcast_convert_type(out, jnp.float16)
    return out_f16.reshape(b, l, EMB_DIM)


# speedup vs baseline: 2.4017x; 2.4017x over previous
import functools

import jax
import jax.numpy as jnp
import numpy as np
from jax import lax
from jax.experimental import pallas as pl
from jax.experimental.pallas import tpu as pltpu
from jax.experimental.pallas import tpu_sc as plsc

VOCAB = 100000
EMB_DIM = 128
GROUP_SIZE = 32
GROUPS = EMB_DIM // GROUP_SIZE
WORDS = EMB_DIM // 4

NUM_CORES = 2
NUM_SUBCORES = 16
NUM_WORKERS = NUM_CORES * NUM_SUBCORES

CHUNK = 512


def _column_perm() -> np.ndarray:
    perm = np.empty(EMB_DIM, dtype=np.int32)
    for p in range(EMB_DIM):
        w, b = p >> 2, p & 3
        c, l = w >> 4, w & 15
        m, o = b >> 1, b & 1
        perm[p] = 64 * c + 32 * m + 2 * l + o
    return perm


_PERM = _column_perm()


def _f16_bits(p):
    b = plsc.bitcast(p, jnp.int32)
    m = b & jnp.int32(0x7FFFFFFF)
    lsb = lax.shift_right_logical(m, jnp.int32(13)) & jnp.int32(1)
    r = (m + lsb) + jnp.int32(0xFFF - 0x38000000)
    h = lax.shift_right_logical(r, jnp.int32(13))
    h = jnp.where(m < jnp.int32(0x38800000), jnp.int32(0), h)
    s = lax.shift_right_logical(b, jnp.int32(16)) & jnp.int32(0x8000)
    return s | h


def _extract_byte(v, b):
    if b == 3:
        return lax.shift_right_arithmetic(v, jnp.int32(24))
    return lax.shift_right_arithmetic(
        lax.shift_left(v, jnp.int32(24 - 8 * b)), jnp.int32(24)
    )


def _make_kernel(n_total: int):
    per_worker = n_total // NUM_WORKERS
    n_chunks = per_worker // CHUNK
    mesh = plsc.VectorSubcoreMesh(core_axis_name="c", subcore_axis_name="s")

    @functools.partial(
        pl.kernel,
        out_type=jax.ShapeDtypeStruct((n_total, EMB_DIM), jnp.int16),
        mesh=mesh,
        scratch_types=[
            pltpu.VMEM((CHUNK,), jnp.int32),
            pltpu.VMEM((CHUNK, WORDS), jnp.int32),
            pltpu.VMEM((CHUNK, GROUPS), jnp.float32),
            pltpu.VMEM((CHUNK, EMB_DIM), jnp.int16),
            pltpu.SemaphoreType.DMA,
            pltpu.SemaphoreType.DMA,
        ],
        compiler_params=pltpu.CompilerParams(
            needs_layout_passes=False, use_tc_tiling_on_sc=False
        ),
    )
    def k(
        idx_hbm, w_hbm, s_hbm, out_hbm,
        idx_v, w_v, s_v, o_v, sem_w, sem_s,
    ):
        wid = lax.axis_index("s") * NUM_CORES + lax.axis_index("c")
        base = wid * per_worker

        def chunk_body(ci, _):
            cb = base + ci * CHUNK
            pltpu.sync_copy(idx_hbm.at[pl.ds(cb, CHUNK)], idx_v)
            cw = pltpu.async_copy(w_hbm.at[idx_v], w_v, sem_w)
            cs = pltpu.async_copy(s_hbm.at[idx_v], s_v, sem_s)
            cw.wait()
            cs.wait()

            def row_body(i, _):
                srow = jnp.full((16,), i, jnp.int32)
                w0 = w_v[i, pl.ds(0, 16)]
                w1 = w_v[i, pl.ds(16, 16)]
                for g in range(GROUPS):
                    vreg = w0 if g < 2 else w1
                    m = g & 1
                    ev = _extract_byte(vreg, 2 * m).astype(jnp.float32)
                    od = _extract_byte(vreg, 2 * m + 1).astype(jnp.float32)
                    sc = plsc.load_gather(
                        s_v, [srow, jnp.full((16,), g, jnp.int32)]
                    )
                    ei = _f16_bits(ev * sc)
                    oi = _f16_bits(od * sc)
                    words = ei | lax.shift_left(oi, jnp.int32(16))
                    o_v[i, pl.ds(32 * g, 32)] = plsc.bitcast(
                        words, jnp.int16
                    )
                return 0

            lax.fori_loop(0, CHUNK, row_body, 0)
            pltpu.sync_copy(o_v, out_hbm.at[pl.ds(cb, CHUNK)])
            return 0

        lax.fori_loop(0, n_chunks, chunk_body, 0)

    return k


@jax.jit
def kernel(indices, weight, scales):
    b, l = indices.shape
    n = b * l
    idx_flat = indices.reshape(n).astype(jnp.int32)
    w_perm = weight.reshape(VOCAB, 2, 2, 16, 2).transpose(0, 1, 3, 2, 4)
    w_i32 = lax.bitcast_convert_type(
        w_perm.reshape(VOCAB, WORDS, 4), jnp.int32
    )
    s_f32 = scales.astype(jnp.float32)
    out = _make_kernel(n)(idx_flat, w_i32, s_f32)
    out_f16 = lax.bitcast_convert_type(out, jnp.float16)
    return out_f16.reshape(b, l, EMB_DIM)
